# baseline (speedup 1.0000x reference)
import functools

import jax
import jax.numpy as jnp
from jax import lax
from jax.experimental import pallas as pl
from jax.experimental.pallas import tpu as pltpu
from jax.experimental.pallas import tpu_sc as plsc

E = 64
IN = 768
HID = 1536
OUT = 768
NB = 8
R = 8
ALPHA = 16.0
K = 2
N = 2048
SCALING = ALPHA / R

BT = 128
MAXB = N * K // BT + E
P = MAXB * BT

NEG = -3.0e38

NC = 2
NS = 16
NW = NC * NS


def _gating_kernel(x_ref, wg_ref, a1_ref, a2_ref, g1_ref, g2_ref,
                   r0_ref, r1_ref, counts_ref, loss_ref):
    x = x_ref[...]
    logits = jnp.dot(x, wg_ref[...], preferred_element_type=jnp.float32)
    iota = lax.broadcasted_iota(jnp.int32, (N, E), 1)
    m1 = jnp.max(logits, axis=1, keepdims=True)
    idx1 = jnp.min(jnp.where(logits == m1, iota, E), axis=1, keepdims=True)
    sel1 = iota == idx1
    l2 = jnp.where(sel1, NEG, logits)
    m2 = jnp.max(l2, axis=1, keepdims=True)
    idx2 = jnp.min(jnp.where(l2 == m2, iota, E), axis=1, keepdims=True)
    sel2 = iota == idx2
    ed = jnp.exp(m2 - m1)
    g1 = 1.0 / (1.0 + ed)
    g2 = ed / (1.0 + ed)

    a1_ref[...] = idx1
    a2_ref[...] = idx2
    g1_ref[...] = g1
    g2_ref[...] = g2

    oh1 = sel1.astype(jnp.float32)
    oh2 = sel2.astype(jnp.float32)

    ri = lax.broadcasted_iota(jnp.int32, (N, N), 0)
    ci = lax.broadcasted_iota(jnp.int32, (N, N), 1)
    lt = (ci < ri).astype(jnp.float32)
    oh = jnp.concatenate([oh1, oh2], axis=1)
    prefix = jnp.dot(lt, oh, preferred_element_type=jnp.float32)
    p1 = prefix[:, :E]
    p2 = prefix[:, E:]
    c1 = jnp.sum(oh1, axis=0, keepdims=True)
    rank0 = jnp.sum(jnp.where(sel1, p1, 0.0), axis=1, keepdims=True)
    rank1 = jnp.sum(jnp.where(sel2, c1 + p2, 0.0), axis=1, keepdims=True)
    r0_ref[...] = rank0.astype(jnp.int32)
    r1_ref[...] = rank1.astype(jnp.int32)
    counts_ref[...] = (c1 + jnp.sum(oh2, axis=0, keepdims=True)).astype(jnp.int32)

    gates = jnp.where(sel1, g1, 0.0) + jnp.where(sel2, g2, 0.0)
    importance = jnp.sum(gates, axis=0)
    load = jnp.sum((gates > 0).astype(jnp.float32), axis=0)

    def cv_sq(v):
        mean = jnp.mean(v)
        var = jnp.sum((v - mean) ** 2) / (E - 1)
        return var / (mean * mean + 1e-10)

    loss_ref[0, 0] = (cv_sq(importance) + cv_sq(load)) * 0.01


def _gmm_kernel(be_ref, xd_ref, bv_ref, gv_ref,
                w1_ref, b1_ref, w2_ref, b2_ref,
                a1_ref, bb1_ref, a2_ref, bb2_ref, out_ref):
    x = xd_ref[...]
    bands = bv_ref[0]
    iota_nbr = lax.broadcasted_iota(jnp.int32, (BT, NB * R), 1)
    mask = (lax.div(iota_nbr, R) == bands).astype(jnp.float32)

    lh = jnp.dot(x, a1_ref[0], preferred_element_type=jnp.float32) * mask
    lh = jnp.dot(lh.astype(jnp.bfloat16), bb1_ref[0],
                 preferred_element_type=jnp.float32)
    h = jnp.dot(x, w1_ref[0], preferred_element_type=jnp.float32)
    h = h + b1_ref[0] + lh * SCALING
    h = h * 0.5 * (1.0 + lax.erf(h * 0.7071067811865476))
    hb = h.astype(jnp.bfloat16)

    lo = jnp.dot(hb, a2_ref[0], preferred_element_type=jnp.float32) * mask
    lo = jnp.dot(lo.astype(jnp.bfloat16), bb2_ref[0],
                 preferred_element_type=jnp.float32)
    out = jnp.dot(hb, w2_ref[0], preferred_element_type=jnp.float32)
    out = out + b2_ref[0] + lo * SCALING
    out_ref[...] = out * gv_ref[0]


_CH = 128
_PW = IN // 2
_RPW = P // NW
_TPW = N // NW


@functools.lru_cache(maxsize=None)
def _build_sc_dispatch():
    nch = _RPW // _CH

    @functools.partial(
        pl.kernel,
        mesh=plsc.VectorSubcoreMesh(core_axis_name="c", subcore_axis_name="s"),
        out_type=jax.ShapeDtypeStruct((P, _PW), jnp.float32),
        scratch_types=[
            pltpu.VMEM((_RPW,), jnp.int32),
            pltpu.VMEM((_CH, _PW), jnp.float32),
            pltpu.VMEM((_CH, _PW), jnp.float32),
            pltpu.SemaphoreType.DMA,
            pltpu.SemaphoreType.DMA,
        ],
    )
    def k(x_hbm, tids_hbm, xd_hbm, idx_v, rows0_v, rows1_v, sem0, sem1):
        wid = lax.axis_index("s") * NC + lax.axis_index("c")
        base = wid * _RPW
        pltpu.sync_copy(tids_hbm.at[pl.ds(base, _RPW)], idx_v)
        bufs = (rows0_v, rows1_v)
        sems = (sem0, sem1)
        cur = pltpu.async_copy(x_hbm.at[idx_v.at[pl.ds(0, _CH)]],
                               bufs[0], sems[0])
        for c in range(nch):
            cur.wait()
            if c + 1 < nch:
                nxt = pltpu.async_copy(
                    x_hbm.at[idx_v.at[pl.ds((c + 1) * _CH, _CH)]],
                    bufs[(c + 1) % 2], sems[(c + 1) % 2])
            pltpu.sync_copy(bufs[c % 2], xd_hbm.at[pl.ds(base + c * _CH, _CH)])
            if c + 1 < nch:
                cur = nxt
    return k


@functools.lru_cache(maxsize=None)
def _build_sc_combine():
    @functools.partial(
        pl.kernel,
        mesh=plsc.VectorSubcoreMesh(core_axis_name="c", subcore_axis_name="s"),
        out_type=jax.ShapeDtypeStruct((N, OUT), jnp.float32),
        scratch_types=[
            pltpu.VMEM((_TPW,), jnp.int32),
            pltpu.VMEM((_TPW,), jnp.int32),
            pltpu.VMEM((_TPW, OUT), jnp.float32),
            pltpu.VMEM((_TPW, OUT), jnp.float32),
            pltpu.SemaphoreType.DMA,
        ],
    )
    def k(outw_hbm, d0_hbm, d1_hbm, y_hbm, i0_v, i1_v, r0_v, r1_v, sem):
        wid = lax.axis_index("s") * NC + lax.axis_index("c")
        base = wid * _TPW
        pltpu.sync_copy(d0_hbm.at[pl.ds(base, _TPW)], i0_v)
        pltpu.sync_copy(d1_hbm.at[pl.ds(base, _TPW)], i1_v)
        pltpu.async_copy(outw_hbm.at[i0_v], r0_v, sem).wait()
        pltpu.async_copy(outw_hbm.at[i1_v], r1_v, sem).wait()

        def body(t, _):
            def cbody(j, _):
                cs = pl.ds(j * 16, 16)
                r0_v[t, cs] = r0_v[t, cs] + r1_v[t, cs]
                return 0
            return lax.fori_loop(0, OUT // 16, cbody, 0)

        lax.fori_loop(0, _TPW, body, 0)
        pltpu.sync_copy(r0_v, y_hbm.at[pl.ds(base, _TPW)])
    return k


def _sc_dispatch(x, tids):
    return _build_sc_dispatch()(x, tids)


def _sc_combine(outw, dest0, dest1):
    return _build_sc_combine()(outw, dest0, dest1)


def kernel(x, band_indices, w_gate, fc1_W, fc1_b, fc2_W, fc2_b,
           lora1_A, lora1_B, lora2_A, lora2_B):
    a1, a2, g1, g2, r0, r1, counts, loss = pl.pallas_call(
        _gating_kernel,
        out_shape=(
            jax.ShapeDtypeStruct((N, 1), jnp.int32),
            jax.ShapeDtypeStruct((N, 1), jnp.int32),
            jax.ShapeDtypeStruct((N, 1), jnp.float32),
            jax.ShapeDtypeStruct((N, 1), jnp.float32),
            jax.ShapeDtypeStruct((N, 1), jnp.int32),
            jax.ShapeDtypeStruct((N, 1), jnp.int32),
            jax.ShapeDtypeStruct((1, E), jnp.int32),
            jax.ShapeDtypeStruct((1, 1), jnp.float32),
        ),
        in_specs=[
            pl.BlockSpec((N, IN), lambda: (0, 0)),
            pl.BlockSpec((IN, E), lambda: (0, 0)),
        ],
        out_specs=(
            pl.BlockSpec((N, 1), lambda: (0, 0)),
            pl.BlockSpec((N, 1), lambda: (0, 0)),
            pl.BlockSpec((N, 1), lambda: (0, 0)),
            pl.BlockSpec((N, 1), lambda: (0, 0)),
            pl.BlockSpec((N, 1), lambda: (0, 0)),
            pl.BlockSpec((N, 1), lambda: (0, 0)),
            pl.BlockSpec((1, E), lambda: (0, 0)),
            pl.BlockSpec(memory_space=pltpu.SMEM),
        ),
    )(x, w_gate)

    counts = counts.reshape(E)
    nb = (counts + (BT - 1)) // BT
    ends = jnp.cumsum(nb)
    block_off = ends - nb
    pad_off = block_off * BT
    total_blocks = ends[E - 1]

    a1f_ = a1.reshape(N)
    a2f_ = a2.reshape(N)
    dest0 = pad_off[a1f_] + r0.reshape(N)
    dest1 = pad_off[a2f_] + r1.reshape(N)

    bj = jnp.arange(MAXB, dtype=jnp.int32)
    be_raw = jnp.sum((ends[None, :] <= bj[:, None]).astype(jnp.int32), axis=1)
    be_last = be_raw[jnp.maximum(total_blocks - 1, 0)]
    block_expert = jnp.where(bj < total_blocks, be_raw, be_last)

    tok = jnp.arange(N, dtype=jnp.int32)
    bands = band_indices.astype(jnp.int32)
    tids = jnp.zeros((P,), jnp.int32).at[dest0].set(tok).at[dest1].set(tok)
    gv = jnp.zeros((P,), jnp.float32).at[dest0].set(g1.reshape(N)).at[dest1].set(g2.reshape(N))
    bv = jnp.zeros((P,), jnp.int32).at[dest0].set(bands).at[dest1].set(bands)

    x_pack = lax.bitcast_convert_type(
        x.astype(jnp.bfloat16).reshape(N, _PW, 2), jnp.float32)
    xd_pack = _sc_dispatch(x_pack, tids)
    xd = lax.bitcast_convert_type(xd_pack, jnp.bfloat16).reshape(P, IN)

    bf = jnp.bfloat16
    a1f = lora1_A.transpose(0, 2, 1, 3).reshape(E, IN, NB * R).astype(bf)
    bb1f = lora1_B.reshape(E, NB * R, HID).astype(bf)
    a2f = lora2_A.transpose(0, 2, 1, 3).reshape(E, HID, NB * R).astype(bf)
    bb2f = lora2_B.reshape(E, NB * R, OUT).astype(bf)
    b1_3d = fc1_b.reshape(E, 1, HID)
    b2_3d = fc2_b.reshape(E, 1, OUT)
    bv3 = bv.reshape(MAXB, BT, 1)
    gv3 = gv.reshape(MAXB, BT, 1)

    grid_spec = pltpu.PrefetchScalarGridSpec(
        num_scalar_prefetch=1,
        grid=(MAXB,),
        in_specs=[
            pl.BlockSpec((BT, IN), lambda i, be: (i, 0)),
            pl.BlockSpec((1, BT, 1), lambda i, be: (i, 0, 0)),
            pl.BlockSpec((1, BT, 1), lambda i, be: (i, 0, 0)),
            pl.BlockSpec((1, IN, HID), lambda i, be: (be[i], 0, 0)),
            pl.BlockSpec((1, 1, HID), lambda i, be: (be[i], 0, 0)),
            pl.BlockSpec((1, HID, OUT), lambda i, be: (be[i], 0, 0)),
            pl.BlockSpec((1, 1, OUT), lambda i, be: (be[i], 0, 0)),
            pl.BlockSpec((1, IN, NB * R), lambda i, be: (be[i], 0, 0)),
            pl.BlockSpec((1, NB * R, HID), lambda i, be: (be[i], 0, 0)),
            pl.BlockSpec((1, HID, NB * R), lambda i, be: (be[i], 0, 0)),
            pl.BlockSpec((1, NB * R, OUT), lambda i, be: (be[i], 0, 0)),
        ],
        out_specs=pl.BlockSpec((BT, OUT), lambda i, be: (i, 0)),
    )
    outw = pl.pallas_call(
        _gmm_kernel,
        grid_spec=grid_spec,
        out_shape=jax.ShapeDtypeStruct((P, OUT), jnp.float32),
    )(block_expert, xd, bv3, gv3, fc1_W.astype(bf), b1_3d, fc2_W.astype(bf),
      b2_3d, a1f, bb1f, a2f, bb2f)

    y = _sc_combine(outw, dest0, dest1)

    return y, loss[0, 0]


# speedup vs baseline: 1.1974x; 1.1974x over previous
import functools

import jax
import jax.numpy as jnp
from jax import lax
from jax.experimental import pallas as pl
from jax.experimental.pallas import tpu as pltpu
from jax.experimental.pallas import tpu_sc as plsc

E = 64
IN = 768
HID = 1536
OUT = 768
NB = 8
R = 8
ALPHA = 16.0
K = 2
N = 2048
SCALING = ALPHA / R

BT = 128
MAXB = N * K // BT + E
P = MAXB * BT

NEG = -3.0e38

NC = 2
NS = 16
NW = NC * NS


def _gating_kernel(x_ref, wg_ref, a1_ref, a2_ref, g1_ref, g2_ref,
                   r0_ref, r1_ref, counts_ref, loss_ref):
    x = x_ref[...]
    logits = jnp.dot(x, wg_ref[...], preferred_element_type=jnp.float32)
    iota = lax.broadcasted_iota(jnp.int32, (N, E), 1)
    m1 = jnp.max(logits, axis=1, keepdims=True)
    idx1 = jnp.min(jnp.where(logits == m1, iota, E), axis=1, keepdims=True)
    sel1 = iota == idx1
    l2 = jnp.where(sel1, NEG, logits)
    m2 = jnp.max(l2, axis=1, keepdims=True)
    idx2 = jnp.min(jnp.where(l2 == m2, iota, E), axis=1, keepdims=True)
    sel2 = iota == idx2
    ed = jnp.exp(m2 - m1)
    g1 = 1.0 / (1.0 + ed)
    g2 = ed / (1.0 + ed)

    a1_ref[...] = idx1
    a2_ref[...] = idx2
    g1_ref[...] = g1
    g2_ref[...] = g2

    oh1 = sel1.astype(jnp.float32)
    oh2 = sel2.astype(jnp.float32)

    ri = lax.broadcasted_iota(jnp.int32, (N, N), 0)
    ci = lax.broadcasted_iota(jnp.int32, (N, N), 1)
    lt = (ci < ri).astype(jnp.float32)
    oh = jnp.concatenate([oh1, oh2], axis=1)
    prefix = jnp.dot(lt, oh, preferred_element_type=jnp.float32)
    p1 = prefix[:, :E]
    p2 = prefix[:, E:]
    c1 = jnp.sum(oh1, axis=0, keepdims=True)
    rank0 = jnp.sum(jnp.where(sel1, p1, 0.0), axis=1, keepdims=True)
    rank1 = jnp.sum(jnp.where(sel2, c1 + p2, 0.0), axis=1, keepdims=True)
    r0_ref[...] = rank0.astype(jnp.int32)
    r1_ref[...] = rank1.astype(jnp.int32)
    counts_ref[...] = (c1 + jnp.sum(oh2, axis=0, keepdims=True)).astype(jnp.int32)

    gates = jnp.where(sel1, g1, 0.0) + jnp.where(sel2, g2, 0.0)
    importance = jnp.sum(gates, axis=0)
    load = jnp.sum((gates > 0).astype(jnp.float32), axis=0)

    def cv_sq(v):
        mean = jnp.mean(v)
        var = jnp.sum((v - mean) ** 2) / (E - 1)
        return var / (mean * mean + 1e-10)

    loss_ref[0, 0] = (cv_sq(importance) + cv_sq(load)) * 0.01


def _gmm_kernel(be_ref, xd_ref, bv_ref, gv_ref,
                w1_ref, b1_ref, w2_ref, b2_ref,
                a1_ref, bb1_ref, a2_ref, bb2_ref, out_ref):
    x = xd_ref[...].astype(jnp.float32)
    bands = bv_ref[0]
    iota_nbr = lax.broadcasted_iota(jnp.int32, (BT, NB * R), 1)
    mask = (lax.div(iota_nbr, R) == bands).astype(jnp.float32)

    lh = jnp.dot(x, a1_ref[0], preferred_element_type=jnp.float32) * mask
    lh = jnp.dot(lh, bb1_ref[0], preferred_element_type=jnp.float32)
    h = jnp.dot(x, w1_ref[0], preferred_element_type=jnp.float32)
    h = h + b1_ref[0] + lh * SCALING
    h = h * 0.5 * (1.0 + lax.erf(h * 0.7071067811865476))

    lo = jnp.dot(h, a2_ref[0], preferred_element_type=jnp.float32) * mask
    lo = jnp.dot(lo, bb2_ref[0], preferred_element_type=jnp.float32)
    out = jnp.dot(h, w2_ref[0], preferred_element_type=jnp.float32)
    out = out + b2_ref[0] + lo * SCALING
    out_ref[...] = out * gv_ref[0]


_CH = 128
_PW = IN // 2
_RPW = P // NW
_TPW = N // NW


@functools.lru_cache(maxsize=None)
def _build_sc_dispatch():
    nch = _RPW // _CH

    @functools.partial(
        pl.kernel,
        mesh=plsc.VectorSubcoreMesh(core_axis_name="c", subcore_axis_name="s"),
        out_type=jax.ShapeDtypeStruct((P, _PW), jnp.float32),
        scratch_types=[
            pltpu.VMEM((_RPW,), jnp.int32),
            pltpu.VMEM((_CH, _PW), jnp.float32),
            pltpu.VMEM((_CH, _PW), jnp.float32),
            pltpu.SemaphoreType.DMA,
            pltpu.SemaphoreType.DMA,
        ],
    )
    def k(x_hbm, tids_hbm, xd_hbm, idx_v, rows0_v, rows1_v, sem0, sem1):
        wid = lax.axis_index("s") * NC + lax.axis_index("c")
        base = wid * _RPW
        pltpu.sync_copy(tids_hbm.at[pl.ds(base, _RPW)], idx_v)
        bufs = (rows0_v, rows1_v)
        sems = (sem0, sem1)
        cur = pltpu.async_copy(x_hbm.at[idx_v.at[pl.ds(0, _CH)]],
                               bufs[0], sems[0])
        for c in range(nch):
            cur.wait()
            if c + 1 < nch:
                nxt = pltpu.async_copy(
                    x_hbm.at[idx_v.at[pl.ds((c + 1) * _CH, _CH)]],
                    bufs[(c + 1) % 2], sems[(c + 1) % 2])
            pltpu.sync_copy(bufs[c % 2], xd_hbm.at[pl.ds(base + c * _CH, _CH)])
            if c + 1 < nch:
                cur = nxt
    return k


@functools.lru_cache(maxsize=None)
def _build_sc_combine():
    @functools.partial(
        pl.kernel,
        mesh=plsc.VectorSubcoreMesh(core_axis_name="c", subcore_axis_name="s"),
        out_type=jax.ShapeDtypeStruct((N, OUT), jnp.float32),
        scratch_types=[
            pltpu.VMEM((_TPW,), jnp.int32),
            pltpu.VMEM((_TPW,), jnp.int32),
            pltpu.VMEM((_TPW, OUT), jnp.float32),
            pltpu.VMEM((_TPW, OUT), jnp.float32),
            pltpu.SemaphoreType.DMA,
        ],
    )
    def k(outw_hbm, d0_hbm, d1_hbm, y_hbm, i0_v, i1_v, r0_v, r1_v, sem):
        wid = lax.axis_index("s") * NC + lax.axis_index("c")
        base = wid * _TPW
        pltpu.sync_copy(d0_hbm.at[pl.ds(base, _TPW)], i0_v)
        pltpu.sync_copy(d1_hbm.at[pl.ds(base, _TPW)], i1_v)
        pltpu.async_copy(outw_hbm.at[i0_v], r0_v, sem).wait()
        pltpu.async_copy(outw_hbm.at[i1_v], r1_v, sem).wait()

        def body(t, _):
            def cbody(j, _):
                cs = pl.ds(j * 16, 16)
                r0_v[t, cs] = r0_v[t, cs] + r1_v[t, cs]
                return 0
            return lax.fori_loop(0, OUT // 16, cbody, 0)

        lax.fori_loop(0, _TPW, body, 0)
        pltpu.sync_copy(r0_v, y_hbm.at[pl.ds(base, _TPW)])
    return k


def _sc_dispatch(x, tids):
    return _build_sc_dispatch()(x, tids)


def _sc_combine(outw, dest0, dest1):
    return _build_sc_combine()(outw, dest0, dest1)


def kernel(x, band_indices, w_gate, fc1_W, fc1_b, fc2_W, fc2_b,
           lora1_A, lora1_B, lora2_A, lora2_B):
    a1, a2, g1, g2, r0, r1, counts, loss = pl.pallas_call(
        _gating_kernel,
        out_shape=(
            jax.ShapeDtypeStruct((N, 1), jnp.int32),
            jax.ShapeDtypeStruct((N, 1), jnp.int32),
            jax.ShapeDtypeStruct((N, 1), jnp.float32),
            jax.ShapeDtypeStruct((N, 1), jnp.float32),
            jax.ShapeDtypeStruct((N, 1), jnp.int32),
            jax.ShapeDtypeStruct((N, 1), jnp.int32),
            jax.ShapeDtypeStruct((1, E), jnp.int32),
            jax.ShapeDtypeStruct((1, 1), jnp.float32),
        ),
        in_specs=[
            pl.BlockSpec((N, IN), lambda: (0, 0)),
            pl.BlockSpec((IN, E), lambda: (0, 0)),
        ],
        out_specs=(
            pl.BlockSpec((N, 1), lambda: (0, 0)),
            pl.BlockSpec((N, 1), lambda: (0, 0)),
            pl.BlockSpec((N, 1), lambda: (0, 0)),
            pl.BlockSpec((N, 1), lambda: (0, 0)),
            pl.BlockSpec((N, 1), lambda: (0, 0)),
            pl.BlockSpec((N, 1), lambda: (0, 0)),
            pl.BlockSpec((1, E), lambda: (0, 0)),
            pl.BlockSpec(memory_space=pltpu.SMEM),
        ),
    )(x, w_gate)

    counts = counts.reshape(E)
    nb = (counts + (BT - 1)) // BT
    ends = jnp.cumsum(nb)
    block_off = ends - nb
    pad_off = block_off * BT
    total_blocks = ends[E - 1]

    a1f_ = a1.reshape(N)
    a2f_ = a2.reshape(N)
    dest0 = pad_off[a1f_] + r0.reshape(N)
    dest1 = pad_off[a2f_] + r1.reshape(N)

    bj = jnp.arange(MAXB, dtype=jnp.int32)
    be_raw = jnp.sum((ends[None, :] <= bj[:, None]).astype(jnp.int32), axis=1)
    be_last = be_raw[jnp.maximum(total_blocks - 1, 0)]
    block_expert = jnp.where(bj < total_blocks, be_raw, be_last)

    tok = jnp.arange(N, dtype=jnp.int32)
    bands = band_indices.astype(jnp.int32)
    tids = jnp.zeros((P,), jnp.int32).at[dest0].set(tok).at[dest1].set(tok)
    gv = jnp.zeros((P,), jnp.float32).at[dest0].set(g1.reshape(N)).at[dest1].set(g2.reshape(N))
    bv = jnp.zeros((P,), jnp.int32).at[dest0].set(bands).at[dest1].set(bands)

    x_pack = lax.bitcast_convert_type(
        x.astype(jnp.bfloat16).reshape(N, _PW, 2), jnp.float32)
    xd_pack = _sc_dispatch(x_pack, tids)
    xd = lax.bitcast_convert_type(xd_pack, jnp.bfloat16).reshape(P, IN)

    a1f = lora1_A.transpose(0, 2, 1, 3).reshape(E, IN, NB * R)
    bb1f = lora1_B.reshape(E, NB * R, HID)
    a2f = lora2_A.transpose(0, 2, 1, 3).reshape(E, HID, NB * R)
    bb2f = lora2_B.reshape(E, NB * R, OUT)
    b1_3d = fc1_b.reshape(E, 1, HID)
    b2_3d = fc2_b.reshape(E, 1, OUT)
    bv3 = bv.reshape(MAXB, BT, 1)
    gv3 = gv.reshape(MAXB, BT, 1)

    grid_spec = pltpu.PrefetchScalarGridSpec(
        num_scalar_prefetch=1,
        grid=(MAXB,),
        in_specs=[
            pl.BlockSpec((BT, IN), lambda i, be: (i, 0)),
            pl.BlockSpec((1, BT, 1), lambda i, be: (i, 0, 0)),
            pl.BlockSpec((1, BT, 1), lambda i, be: (i, 0, 0)),
            pl.BlockSpec((1, IN, HID), lambda i, be: (be[i], 0, 0)),
            pl.BlockSpec((1, 1, HID), lambda i, be: (be[i], 0, 0)),
            pl.BlockSpec((1, HID, OUT), lambda i, be: (be[i], 0, 0)),
            pl.BlockSpec((1, 1, OUT), lambda i, be: (be[i], 0, 0)),
            pl.BlockSpec((1, IN, NB * R), lambda i, be: (be[i], 0, 0)),
            pl.BlockSpec((1, NB * R, HID), lambda i, be: (be[i], 0, 0)),
            pl.BlockSpec((1, HID, NB * R), lambda i, be: (be[i], 0, 0)),
            pl.BlockSpec((1, NB * R, OUT), lambda i, be: (be[i], 0, 0)),
        ],
        out_specs=pl.BlockSpec((BT, OUT), lambda i, be: (i, 0)),
    )
    outw = pl.pallas_call(
        _gmm_kernel,
        grid_spec=grid_spec,
        out_shape=jax.ShapeDtypeStruct((P, OUT), jnp.float32),
    )(block_expert, xd, bv3, gv3, fc1_W, b1_3d, fc2_W, b2_3d,
      a1f, bb1f, a2f, bb2f)

    y = _sc_combine(outw, dest0, dest1)

    return y, loss[0, 0]


# speedup vs baseline: 1.5592x; 1.3021x over previous
import functools

import jax
import jax.numpy as jnp
from jax import lax
from jax.experimental import pallas as pl
from jax.experimental.pallas import tpu as pltpu
from jax.experimental.pallas import tpu_sc as plsc

E = 64
IN = 768
HID = 1536
OUT = 768
NB = 8
R = 8
ALPHA = 16.0
K = 2
N = 2048
SCALING = ALPHA / R

BT = 128
MAXB = N * K // BT + E
P = MAXB * BT

NEG = -3.0e38

NC = 2
NS = 16
NW = NC * NS


def _gating_kernel(x_ref, wg_ref, a1_ref, a2_ref, g1_ref, g2_ref,
                   r0_ref, r1_ref, counts_ref, loss_ref):
    x = x_ref[...]
    logits = jnp.dot(x, wg_ref[...], preferred_element_type=jnp.float32)
    iota = lax.broadcasted_iota(jnp.int32, (N, E), 1)
    m1 = jnp.max(logits, axis=1, keepdims=True)
    idx1 = jnp.min(jnp.where(logits == m1, iota, E), axis=1, keepdims=True)
    sel1 = iota == idx1
    l2 = jnp.where(sel1, NEG, logits)
    m2 = jnp.max(l2, axis=1, keepdims=True)
    idx2 = jnp.min(jnp.where(l2 == m2, iota, E), axis=1, keepdims=True)
    sel2 = iota == idx2
    ed = jnp.exp(m2 - m1)
    g1 = 1.0 / (1.0 + ed)
    g2 = ed / (1.0 + ed)

    a1_ref[...] = idx1
    a2_ref[...] = idx2
    g1_ref[...] = g1
    g2_ref[...] = g2

    oh1 = sel1.astype(jnp.float32)
    oh2 = sel2.astype(jnp.float32)

    ri = lax.broadcasted_iota(jnp.int32, (N, N), 0)
    ci = lax.broadcasted_iota(jnp.int32, (N, N), 1)
    lt = (ci < ri).astype(jnp.float32)
    oh = jnp.concatenate([oh1, oh2], axis=1)
    prefix = jnp.dot(lt, oh, preferred_element_type=jnp.float32)
    p1 = prefix[:, :E]
    p2 = prefix[:, E:]
    c1 = jnp.sum(oh1, axis=0, keepdims=True)
    rank0 = jnp.sum(jnp.where(sel1, p1, 0.0), axis=1, keepdims=True)
    rank1 = jnp.sum(jnp.where(sel2, c1 + p2, 0.0), axis=1, keepdims=True)
    r0_ref[...] = rank0.astype(jnp.int32)
    r1_ref[...] = rank1.astype(jnp.int32)
    counts_ref[...] = (c1 + jnp.sum(oh2, axis=0, keepdims=True)).astype(jnp.int32)

    gates = jnp.where(sel1, g1, 0.0) + jnp.where(sel2, g2, 0.0)
    importance = jnp.sum(gates, axis=0)
    load = jnp.sum((gates > 0).astype(jnp.float32), axis=0)

    def cv_sq(v):
        mean = jnp.mean(v)
        var = jnp.sum((v - mean) ** 2) / (E - 1)
        return var / (mean * mean + 1e-10)

    loss_ref[0, 0] = (cv_sq(importance) + cv_sq(load)) * 0.01


def _gmm_kernel(be_ref, xd_ref, bv_ref, gv_ref,
                w1_ref, b1_ref, w2_ref, b2_ref,
                a1_ref, bb1_ref, a2_ref, bb2_ref, out_ref):
    u = lax.bitcast_convert_type(xd_ref[...], jnp.uint32)
    xa = lax.bitcast_convert_type(u << 16, jnp.float32)
    xb = lax.bitcast_convert_type((u >> 16) << 16, jnp.float32)
    x = jnp.concatenate([xa, xb], axis=1)
    bands = bv_ref[0]
    iota_nbr = lax.broadcasted_iota(jnp.int32, (BT, NB * R), 1)
    mask = (lax.div(iota_nbr, R) == bands).astype(jnp.float32)

    lh = jnp.dot(x, a1_ref[0], preferred_element_type=jnp.float32) * mask
    lh = jnp.dot(lh, bb1_ref[0], preferred_element_type=jnp.float32)
    h = jnp.dot(x, w1_ref[0], preferred_element_type=jnp.float32)
    h = h + b1_ref[0] + lh * SCALING
    h = h * 0.5 * (1.0 + lax.erf(h * 0.7071067811865476))

    lo = jnp.dot(h, a2_ref[0], preferred_element_type=jnp.float32) * mask
    lo = jnp.dot(lo, bb2_ref[0], preferred_element_type=jnp.float32)
    out = jnp.dot(h, w2_ref[0], preferred_element_type=jnp.float32)
    out = out + b2_ref[0] + lo * SCALING
    out_ref[...] = out * gv_ref[0]


_CH = 64
_PW = IN // 2
_RPW = P // NW
_TPW = N // NW


@functools.lru_cache(maxsize=None)
def _build_sc_dispatch():
    nch = _RPW // _CH

    nbuf = 4

    @functools.partial(
        pl.kernel,
        mesh=plsc.VectorSubcoreMesh(core_axis_name="c", subcore_axis_name="s"),
        out_type=jax.ShapeDtypeStruct((P, _PW), jnp.float32),
        scratch_types=(
            [pltpu.VMEM((_RPW,), jnp.int32)]
            + [pltpu.VMEM((_CH, _PW), jnp.float32) for _ in range(nbuf)]
            + [pltpu.SemaphoreType.DMA for _ in range(nbuf)]
        ),
    )
    def k(x_hbm, tids_hbm, xd_hbm, idx_v, *bufs_sems):
        bufs = bufs_sems[:nbuf]
        sems = bufs_sems[nbuf:]
        wid = lax.axis_index("s") * NC + lax.axis_index("c")
        base = wid * _RPW
        pltpu.sync_copy(tids_hbm.at[pl.ds(base, _RPW)], idx_v)

        def gather(c):
            return pltpu.async_copy(
                x_hbm.at[idx_v.at[pl.ds(c * _CH, _CH)]],
                bufs[c % nbuf], sems[c % nbuf])

        pending = [gather(c) for c in range(min(nbuf, nch))]
        for c in range(nch):
            pending[c % nbuf].wait()
            pltpu.sync_copy(bufs[c % nbuf],
                            xd_hbm.at[pl.ds(base + c * _CH, _CH)])
            if c + nbuf < nch:
                pending[c % nbuf] = gather(c + nbuf)
    return k


@functools.lru_cache(maxsize=None)
def _build_sc_combine():
    @functools.partial(
        pl.kernel,
        mesh=plsc.VectorSubcoreMesh(core_axis_name="c", subcore_axis_name="s"),
        out_type=jax.ShapeDtypeStruct((N, OUT), jnp.float32),
        scratch_types=[
            pltpu.VMEM((_TPW,), jnp.int32),
            pltpu.VMEM((_TPW,), jnp.int32),
            pltpu.VMEM((_TPW, OUT), jnp.float32),
            pltpu.VMEM((_TPW, OUT), jnp.float32),
            pltpu.SemaphoreType.DMA,
        ],
    )
    def k(outw_hbm, d0_hbm, d1_hbm, y_hbm, i0_v, i1_v, r0_v, r1_v, sem):
        wid = lax.axis_index("s") * NC + lax.axis_index("c")
        base = wid * _TPW
        pltpu.sync_copy(d0_hbm.at[pl.ds(base, _TPW)], i0_v)
        pltpu.sync_copy(d1_hbm.at[pl.ds(base, _TPW)], i1_v)
        pltpu.async_copy(outw_hbm.at[i0_v], r0_v, sem).wait()
        pltpu.async_copy(outw_hbm.at[i1_v], r1_v, sem).wait()

        def body(t, _):
            def cbody(j, _):
                cs = pl.ds(j * 16, 16)
                r0_v[t, cs] = r0_v[t, cs] + r1_v[t, cs]
                return 0
            return lax.fori_loop(0, OUT // 16, cbody, 0)

        lax.fori_loop(0, _TPW, body, 0)
        pltpu.sync_copy(r0_v, y_hbm.at[pl.ds(base, _TPW)])
    return k


def _sc_dispatch(x, tids):
    return _build_sc_dispatch()(x, tids)


def _sc_combine(outw, dest0, dest1):
    return _build_sc_combine()(outw, dest0, dest1)


def kernel(x, band_indices, w_gate, fc1_W, fc1_b, fc2_W, fc2_b,
           lora1_A, lora1_B, lora2_A, lora2_B):
    a1, a2, g1, g2, r0, r1, counts, loss = pl.pallas_call(
        _gating_kernel,
        out_shape=(
            jax.ShapeDtypeStruct((N, 1), jnp.int32),
            jax.ShapeDtypeStruct((N, 1), jnp.int32),
            jax.ShapeDtypeStruct((N, 1), jnp.float32),
            jax.ShapeDtypeStruct((N, 1), jnp.float32),
            jax.ShapeDtypeStruct((N, 1), jnp.int32),
            jax.ShapeDtypeStruct((N, 1), jnp.int32),
            jax.ShapeDtypeStruct((1, E), jnp.int32),
            jax.ShapeDtypeStruct((1, 1), jnp.float32),
        ),
        in_specs=[
            pl.BlockSpec((N, IN), lambda: (0, 0)),
            pl.BlockSpec((IN, E), lambda: (0, 0)),
        ],
        out_specs=(
            pl.BlockSpec((N, 1), lambda: (0, 0)),
            pl.BlockSpec((N, 1), lambda: (0, 0)),
            pl.BlockSpec((N, 1), lambda: (0, 0)),
            pl.BlockSpec((N, 1), lambda: (0, 0)),
            pl.BlockSpec((N, 1), lambda: (0, 0)),
            pl.BlockSpec((N, 1), lambda: (0, 0)),
            pl.BlockSpec((1, E), lambda: (0, 0)),
            pl.BlockSpec(memory_space=pltpu.SMEM),
        ),
    )(x, w_gate)

    counts = counts.reshape(E)
    nb = (counts + (BT - 1)) // BT
    ends = jnp.cumsum(nb)
    block_off = ends - nb
    pad_off = block_off * BT
    total_blocks = ends[E - 1]

    a1f_ = a1.reshape(N)
    a2f_ = a2.reshape(N)
    dest0 = pad_off[a1f_] + r0.reshape(N)
    dest1 = pad_off[a2f_] + r1.reshape(N)

    bj = jnp.arange(MAXB, dtype=jnp.int32)
    be_raw = jnp.sum((ends[None, :] <= bj[:, None]).astype(jnp.int32), axis=1)
    be_last = be_raw[jnp.maximum(total_blocks - 1, 0)]
    block_expert = jnp.where(bj < total_blocks, be_raw, be_last)

    tok = jnp.arange(N, dtype=jnp.int32)
    bands = band_indices.astype(jnp.int32)
    tids = jnp.zeros((P,), jnp.int32).at[dest0].set(tok).at[dest1].set(tok)
    gv = jnp.zeros((P,), jnp.float32).at[dest0].set(g1.reshape(N)).at[dest1].set(g2.reshape(N))
    bv = jnp.zeros((P,), jnp.int32).at[dest0].set(bands).at[dest1].set(bands)

    x_bf = x.astype(jnp.bfloat16)
    x_pack = lax.bitcast_convert_type(
        jnp.stack([x_bf[:, :_PW], x_bf[:, _PW:]], axis=-1), jnp.float32)
    xd = _sc_dispatch(x_pack, tids)

    a1f = lora1_A.transpose(0, 2, 1, 3).reshape(E, IN, NB * R)
    bb1f = lora1_B.reshape(E, NB * R, HID)
    a2f = lora2_A.transpose(0, 2, 1, 3).reshape(E, HID, NB * R)
    bb2f = lora2_B.reshape(E, NB * R, OUT)
    b1_3d = fc1_b.reshape(E, 1, HID)
    b2_3d = fc2_b.reshape(E, 1, OUT)
    bv3 = bv.reshape(MAXB, BT, 1)
    gv3 = gv.reshape(MAXB, BT, 1)

    grid_spec = pltpu.PrefetchScalarGridSpec(
        num_scalar_prefetch=1,
        grid=(MAXB,),
        in_specs=[
            pl.BlockSpec((BT, _PW), lambda i, be: (i, 0)),
            pl.BlockSpec((1, BT, 1), lambda i, be: (i, 0, 0)),
            pl.BlockSpec((1, BT, 1), lambda i, be: (i, 0, 0)),
            pl.BlockSpec((1, IN, HID), lambda i, be: (be[i], 0, 0)),
            pl.BlockSpec((1, 1, HID), lambda i, be: (be[i], 0, 0)),
            pl.BlockSpec((1, HID, OUT), lambda i, be: (be[i], 0, 0)),
            pl.BlockSpec((1, 1, OUT), lambda i, be: (be[i], 0, 0)),
            pl.BlockSpec((1, IN, NB * R), lambda i, be: (be[i], 0, 0)),
            pl.BlockSpec((1, NB * R, HID), lambda i, be: (be[i], 0, 0)),
            pl.BlockSpec((1, HID, NB * R), lambda i, be: (be[i], 0, 0)),
            pl.BlockSpec((1, NB * R, OUT), lambda i, be: (be[i], 0, 0)),
        ],
        out_specs=pl.BlockSpec((BT, OUT), lambda i, be: (i, 0)),
    )
    outw = pl.pallas_call(
        _gmm_kernel,
        grid_spec=grid_spec,
        out_shape=jax.ShapeDtypeStruct((P, OUT), jnp.float32),
    )(block_expert, xd, bv3, gv3, fc1_W, b1_3d, fc2_W, b2_3d,
      a1f, bb1f, a2f, bb2f)

    y = _sc_combine(outw, dest0, dest1)

    return y, loss[0, 0]


# speedup vs baseline: 2.3644x; 1.5165x over previous
import functools

import jax
import jax.numpy as jnp
from jax import lax
from jax.experimental import pallas as pl
from jax.experimental.pallas import tpu as pltpu
from jax.experimental.pallas import tpu_sc as plsc

E = 64
IN = 768
HID = 1536
OUT = 768
NB = 8
R = 8
ALPHA = 16.0
K = 2
N = 2048
SCALING = ALPHA / R

BT = 128
MAXB = N * K // BT + E
P = MAXB * BT

NEG = -3.0e38

NC = 2
NS = 16
NW = NC * NS


def _gating_kernel(x_ref, wg_ref, a1_ref, a2_ref, g1_ref, g2_ref,
                   r0_ref, r1_ref, counts_ref, loss_ref):
    x = x_ref[...]
    logits = jnp.dot(x, wg_ref[...], preferred_element_type=jnp.float32)
    iota = lax.broadcasted_iota(jnp.int32, (N, E), 1)
    m1 = jnp.max(logits, axis=1, keepdims=True)
    idx1 = jnp.min(jnp.where(logits == m1, iota, E), axis=1, keepdims=True)
    sel1 = iota == idx1
    l2 = jnp.where(sel1, NEG, logits)
    m2 = jnp.max(l2, axis=1, keepdims=True)
    idx2 = jnp.min(jnp.where(l2 == m2, iota, E), axis=1, keepdims=True)
    sel2 = iota == idx2
    ed = jnp.exp(m2 - m1)
    g1 = 1.0 / (1.0 + ed)
    g2 = ed / (1.0 + ed)

    a1_ref[...] = idx1
    a2_ref[...] = idx2
    g1_ref[...] = g1
    g2_ref[...] = g2

    oh1 = sel1.astype(jnp.float32)
    oh2 = sel2.astype(jnp.float32)

    ri = lax.broadcasted_iota(jnp.int32, (N, N), 0)
    ci = lax.broadcasted_iota(jnp.int32, (N, N), 1)
    lt = (ci < ri).astype(jnp.float32)
    oh = jnp.concatenate([oh1, oh2], axis=1)
    prefix = jnp.dot(lt, oh, preferred_element_type=jnp.float32)
    p1 = prefix[:, :E]
    p2 = prefix[:, E:]
    c1 = jnp.sum(oh1, axis=0, keepdims=True)
    rank0 = jnp.sum(jnp.where(sel1, p1, 0.0), axis=1, keepdims=True)
    rank1 = jnp.sum(jnp.where(sel2, c1 + p2, 0.0), axis=1, keepdims=True)
    r0_ref[...] = rank0.astype(jnp.int32)
    r1_ref[...] = rank1.astype(jnp.int32)
    counts_ref[...] = (c1 + jnp.sum(oh2, axis=0, keepdims=True)).astype(jnp.int32)

    gates = jnp.where(sel1, g1, 0.0) + jnp.where(sel2, g2, 0.0)
    importance = jnp.sum(gates, axis=0)
    load = jnp.sum((gates > 0).astype(jnp.float32), axis=0)

    def cv_sq(v):
        mean = jnp.mean(v)
        var = jnp.sum((v - mean) ** 2) / (E - 1)
        return var / (mean * mean + 1e-10)

    loss_ref[0, 0] = (cv_sq(importance) + cv_sq(load)) * 0.01


def _gmm_kernel(be_ref, xd_ref, bv_ref, gv_ref,
                w1_ref, b1_ref, w2_ref, b2_ref,
                a1_ref, bb1_ref, a2_ref, bb2_ref, out_ref):
    u = lax.bitcast_convert_type(xd_ref[...], jnp.uint32)
    xa = lax.bitcast_convert_type(u << 16, jnp.float32)
    xb = lax.bitcast_convert_type((u >> 16) << 16, jnp.float32)
    x = jnp.concatenate([xa, xb], axis=1)
    bands = bv_ref[0]
    iota_nbr = lax.broadcasted_iota(jnp.int32, (BT, NB * R), 1)
    mask = (lax.div(iota_nbr, R) == bands).astype(jnp.float32)

    lh = jnp.dot(x, a1_ref[0], preferred_element_type=jnp.float32) * mask
    lh = jnp.dot(lh, bb1_ref[0], preferred_element_type=jnp.float32)
    h = jnp.dot(x, w1_ref[0], preferred_element_type=jnp.float32)
    h = h + b1_ref[0] + lh * SCALING
    h = h * 0.5 * (1.0 + lax.erf(h * 0.7071067811865476))

    lo = jnp.dot(h, a2_ref[0], preferred_element_type=jnp.float32) * mask
    lo = jnp.dot(lo, bb2_ref[0], preferred_element_type=jnp.float32)
    out = jnp.dot(h, w2_ref[0], preferred_element_type=jnp.float32)
    out = out + b2_ref[0] + lo * SCALING
    out_ref[...] = out * gv_ref[0]


_CH = 64
_PW = IN // 2
_RPW = P // NW
_TPW = N // NW


@functools.lru_cache(maxsize=None)
def _build_sc_dispatch():
    @functools.partial(
        pl.kernel,
        mesh=plsc.VectorSubcoreMesh(core_axis_name="c", subcore_axis_name="s"),
        out_type=jax.ShapeDtypeStruct((P, _PW), jnp.float32),
        scratch_types=[
            pltpu.VMEM((_TPW,), jnp.int32),
            pltpu.VMEM((_TPW,), jnp.int32),
            pltpu.VMEM((_TPW, _PW), jnp.float32),
            pltpu.SemaphoreType.DMA,
            pltpu.SemaphoreType.DMA,
        ],
    )
    def k(x_hbm, d0_hbm, d1_hbm, xd_hbm, i0_v, i1_v, rows_v, sem0, sem1):
        wid = lax.axis_index("s") * NC + lax.axis_index("c")
        base = wid * _TPW
        pltpu.sync_copy(d0_hbm.at[pl.ds(base, _TPW)], i0_v)
        pltpu.sync_copy(d1_hbm.at[pl.ds(base, _TPW)], i1_v)
        pltpu.sync_copy(x_hbm.at[pl.ds(base, _TPW)], rows_v)
        c0 = pltpu.async_copy(rows_v, xd_hbm.at[i0_v], sem0)
        c1 = pltpu.async_copy(rows_v, xd_hbm.at[i1_v], sem1)
        c0.wait()
        c1.wait()
    return k


@functools.lru_cache(maxsize=None)
def _build_sc_combine():
    @functools.partial(
        pl.kernel,
        mesh=plsc.VectorSubcoreMesh(core_axis_name="c", subcore_axis_name="s"),
        out_type=jax.ShapeDtypeStruct((N, OUT), jnp.float32),
        scratch_types=[
            pltpu.VMEM((_TPW,), jnp.int32),
            pltpu.VMEM((_TPW,), jnp.int32),
            pltpu.VMEM((_TPW, OUT), jnp.float32),
            pltpu.VMEM((_TPW, OUT), jnp.float32),
            pltpu.SemaphoreType.DMA,
        ],
    )
    def k(outw_hbm, d0_hbm, d1_hbm, y_hbm, i0_v, i1_v, r0_v, r1_v, sem):
        wid = lax.axis_index("s") * NC + lax.axis_index("c")
        base = wid * _TPW
        pltpu.sync_copy(d0_hbm.at[pl.ds(base, _TPW)], i0_v)
        pltpu.sync_copy(d1_hbm.at[pl.ds(base, _TPW)], i1_v)
        pltpu.async_copy(outw_hbm.at[i0_v], r0_v, sem).wait()
        pltpu.async_copy(outw_hbm.at[i1_v], r1_v, sem).wait()

        def body(t, _):
            def cbody(j, _):
                cs = pl.ds(j * 16, 16)
                r0_v[t, cs] = r0_v[t, cs] + r1_v[t, cs]
                return 0
            return lax.fori_loop(0, OUT // 16, cbody, 0)

        lax.fori_loop(0, _TPW, body, 0)
        pltpu.sync_copy(r0_v, y_hbm.at[pl.ds(base, _TPW)])
    return k


def _sc_dispatch(x, dest0, dest1):
    return _build_sc_dispatch()(x, dest0, dest1)


def _sc_combine(outw, dest0, dest1):
    return _build_sc_combine()(outw, dest0, dest1)


def kernel(x, band_indices, w_gate, fc1_W, fc1_b, fc2_W, fc2_b,
           lora1_A, lora1_B, lora2_A, lora2_B):
    a1, a2, g1, g2, r0, r1, counts, loss = pl.pallas_call(
        _gating_kernel,
        out_shape=(
            jax.ShapeDtypeStruct((N, 1), jnp.int32),
            jax.ShapeDtypeStruct((N, 1), jnp.int32),
            jax.ShapeDtypeStruct((N, 1), jnp.float32),
            jax.ShapeDtypeStruct((N, 1), jnp.float32),
            jax.ShapeDtypeStruct((N, 1), jnp.int32),
            jax.ShapeDtypeStruct((N, 1), jnp.int32),
            jax.ShapeDtypeStruct((1, E), jnp.int32),
            jax.ShapeDtypeStruct((1, 1), jnp.float32),
        ),
        in_specs=[
            pl.BlockSpec((N, IN), lambda: (0, 0)),
            pl.BlockSpec((IN, E), lambda: (0, 0)),
        ],
        out_specs=(
            pl.BlockSpec((N, 1), lambda: (0, 0)),
            pl.BlockSpec((N, 1), lambda: (0, 0)),
            pl.BlockSpec((N, 1), lambda: (0, 0)),
            pl.BlockSpec((N, 1), lambda: (0, 0)),
            pl.BlockSpec((N, 1), lambda: (0, 0)),
            pl.BlockSpec((N, 1), lambda: (0, 0)),
            pl.BlockSpec((1, E), lambda: (0, 0)),
            pl.BlockSpec(memory_space=pltpu.SMEM),
        ),
    )(x, w_gate)

    counts = counts.reshape(E)
    nb = (counts + (BT - 1)) // BT
    ends = jnp.cumsum(nb)
    block_off = ends - nb
    pad_off = block_off * BT
    total_blocks = ends[E - 1]

    a1f_ = a1.reshape(N)
    a2f_ = a2.reshape(N)
    dest0 = pad_off[a1f_] + r0.reshape(N)
    dest1 = pad_off[a2f_] + r1.reshape(N)

    bj = jnp.arange(MAXB, dtype=jnp.int32)
    be_raw = jnp.sum((ends[None, :] <= bj[:, None]).astype(jnp.int32), axis=1)
    be_last = be_raw[jnp.maximum(total_blocks - 1, 0)]
    block_expert = jnp.where(bj < total_blocks, be_raw, be_last)

    bands = band_indices.astype(jnp.int32)
    gv = jnp.zeros((P,), jnp.float32).at[dest0].set(g1.reshape(N)).at[dest1].set(g2.reshape(N))
    bv = jnp.zeros((P,), jnp.int32).at[dest0].set(bands).at[dest1].set(bands)

    x_bf = x.astype(jnp.bfloat16)
    x_pack = lax.bitcast_convert_type(
        jnp.stack([x_bf[:, :_PW], x_bf[:, _PW:]], axis=-1), jnp.float32)
    xd = _sc_dispatch(x_pack, dest0, dest1)

    a1f = lora1_A.transpose(0, 2, 1, 3).reshape(E, IN, NB * R)
    bb1f = lora1_B.reshape(E, NB * R, HID)
    a2f = lora2_A.transpose(0, 2, 1, 3).reshape(E, HID, NB * R)
    bb2f = lora2_B.reshape(E, NB * R, OUT)
    b1_3d = fc1_b.reshape(E, 1, HID)
    b2_3d = fc2_b.reshape(E, 1, OUT)
    bv3 = bv.reshape(MAXB, BT, 1)
    gv3 = gv.reshape(MAXB, BT, 1)

    grid_spec = pltpu.PrefetchScalarGridSpec(
        num_scalar_prefetch=1,
        grid=(MAXB,),
        in_specs=[
            pl.BlockSpec((BT, _PW), lambda i, be: (i, 0)),
            pl.BlockSpec((1, BT, 1), lambda i, be: (i, 0, 0)),
            pl.BlockSpec((1, BT, 1), lambda i, be: (i, 0, 0)),
            pl.BlockSpec((1, IN, HID), lambda i, be: (be[i], 0, 0)),
            pl.BlockSpec((1, 1, HID), lambda i, be: (be[i], 0, 0)),
            pl.BlockSpec((1, HID, OUT), lambda i, be: (be[i], 0, 0)),
            pl.BlockSpec((1, 1, OUT), lambda i, be: (be[i], 0, 0)),
            pl.BlockSpec((1, IN, NB * R), lambda i, be: (be[i], 0, 0)),
            pl.BlockSpec((1, NB * R, HID), lambda i, be: (be[i], 0, 0)),
            pl.BlockSpec((1, HID, NB * R), lambda i, be: (be[i], 0, 0)),
            pl.BlockSpec((1, NB * R, OUT), lambda i, be: (be[i], 0, 0)),
        ],
        out_specs=pl.BlockSpec((BT, OUT), lambda i, be: (i, 0)),
    )
    outw = pl.pallas_call(
        _gmm_kernel,
        grid_spec=grid_spec,
        out_shape=jax.ShapeDtypeStruct((P, OUT), jnp.float32),
    )(block_expert, xd, bv3, gv3, fc1_W, b1_3d, fc2_W, b2_3d,
      a1f, bb1f, a2f, bb2f)

    y = _sc_combine(outw, dest0, dest1)

    return y, loss[0, 0]


# speedup vs baseline: 2.5319x; 1.0708x over previous
import functools

import jax
import jax.numpy as jnp
from jax import lax
from jax.experimental import pallas as pl
from jax.experimental.pallas import tpu as pltpu
from jax.experimental.pallas import tpu_sc as plsc

E = 64
IN = 768
HID = 1536
OUT = 768
NB = 8
R = 8
ALPHA = 16.0
K = 2
N = 2048
SCALING = ALPHA / R

BT = 128
MAXB = N * K // BT + E
P = MAXB * BT

NEG = -3.0e38

NC = 2
NS = 16
NW = NC * NS


def _gating_kernel(x_ref, wg_ref, a1_ref, a2_ref, g1_ref, g2_ref,
                   r0_ref, r1_ref, counts_ref, loss_ref):
    x = x_ref[...]
    logits = jnp.dot(x, wg_ref[...], preferred_element_type=jnp.float32)
    iota = lax.broadcasted_iota(jnp.int32, (N, E), 1)
    m1 = jnp.max(logits, axis=1, keepdims=True)
    idx1 = jnp.min(jnp.where(logits == m1, iota, E), axis=1, keepdims=True)
    sel1 = iota == idx1
    l2 = jnp.where(sel1, NEG, logits)
    m2 = jnp.max(l2, axis=1, keepdims=True)
    idx2 = jnp.min(jnp.where(l2 == m2, iota, E), axis=1, keepdims=True)
    sel2 = iota == idx2
    ed = jnp.exp(m2 - m1)
    g1 = 1.0 / (1.0 + ed)
    g2 = ed / (1.0 + ed)

    a1_ref[...] = idx1
    a2_ref[...] = idx2
    g1_ref[...] = g1
    g2_ref[...] = g2

    oh1 = sel1.astype(jnp.float32)
    oh2 = sel2.astype(jnp.float32)

    ri = lax.broadcasted_iota(jnp.int32, (N, N), 0)
    ci = lax.broadcasted_iota(jnp.int32, (N, N), 1)
    lt = (ci < ri).astype(jnp.float32)
    oh = jnp.concatenate([oh1, oh2], axis=1)
    prefix = jnp.dot(lt, oh, preferred_element_type=jnp.float32)
    p1 = prefix[:, :E]
    p2 = prefix[:, E:]
    c1 = jnp.sum(oh1, axis=0, keepdims=True)
    rank0 = jnp.sum(jnp.where(sel1, p1, 0.0), axis=1, keepdims=True)
    rank1 = jnp.sum(jnp.where(sel2, c1 + p2, 0.0), axis=1, keepdims=True)
    r0_ref[...] = rank0.astype(jnp.int32)
    r1_ref[...] = rank1.astype(jnp.int32)
    counts_ref[...] = (c1 + jnp.sum(oh2, axis=0, keepdims=True)).astype(jnp.int32)

    gates = jnp.where(sel1, g1, 0.0) + jnp.where(sel2, g2, 0.0)
    importance = jnp.sum(gates, axis=0)
    load = jnp.sum((gates > 0).astype(jnp.float32), axis=0)

    def cv_sq(v):
        mean = jnp.mean(v)
        var = jnp.sum((v - mean) ** 2) / (E - 1)
        return var / (mean * mean + 1e-10)

    loss_ref[0, 0] = (cv_sq(importance) + cv_sq(load)) * 0.01


def _gmm_kernel(be_ref, nb_ref, xd_ref, bv_ref, gv_ref,
                w1_ref, b1_ref, w2_ref, b2_ref,
                a1_ref, bb1_ref, a2_ref, bb2_ref, out_ref):
    @pl.when(pl.program_id(0) < nb_ref[0])
    def _():
        u = lax.bitcast_convert_type(xd_ref[...], jnp.uint32)
        xa = lax.bitcast_convert_type(u << 16, jnp.float32)
        xb = lax.bitcast_convert_type((u >> 16) << 16, jnp.float32)
        x = jnp.concatenate([xa, xb], axis=1)
        bands = bv_ref[0]
        iota_nbr = lax.broadcasted_iota(jnp.int32, (BT, NB * R), 1)
        mask = (lax.div(iota_nbr, R) == bands).astype(jnp.float32)

        lh = jnp.dot(x, a1_ref[0], preferred_element_type=jnp.float32) * mask
        lh = jnp.dot(lh, bb1_ref[0], preferred_element_type=jnp.float32)
        h = jnp.dot(x, w1_ref[0], preferred_element_type=jnp.float32)
        h = h + b1_ref[0] + lh * SCALING
        h = h * 0.5 * (1.0 + lax.erf(h * 0.7071067811865476))

        lo = jnp.dot(h, a2_ref[0], preferred_element_type=jnp.float32) * mask
        lo = jnp.dot(lo, bb2_ref[0], preferred_element_type=jnp.float32)
        out = jnp.dot(h, w2_ref[0], preferred_element_type=jnp.float32)
        out = out + b2_ref[0] + lo * SCALING
        out_ref[...] = out * gv_ref[0]


_CH = 64
_PW = IN // 2
_RPW = P // NW
_TPW = N // NW


@functools.lru_cache(maxsize=None)
def _build_sc_dispatch():
    @functools.partial(
        pl.kernel,
        mesh=plsc.VectorSubcoreMesh(core_axis_name="c", subcore_axis_name="s"),
        out_type=jax.ShapeDtypeStruct((P, _PW), jnp.float32),
        scratch_types=[
            pltpu.VMEM((_TPW,), jnp.int32),
            pltpu.VMEM((_TPW,), jnp.int32),
            pltpu.VMEM((_TPW, _PW), jnp.float32),
            pltpu.SemaphoreType.DMA,
            pltpu.SemaphoreType.DMA,
        ],
    )
    def k(x_hbm, d0_hbm, d1_hbm, xd_hbm, i0_v, i1_v, rows_v, sem0, sem1):
        wid = lax.axis_index("s") * NC + lax.axis_index("c")
        base = wid * _TPW
        pltpu.sync_copy(d0_hbm.at[pl.ds(base, _TPW)], i0_v)
        pltpu.sync_copy(d1_hbm.at[pl.ds(base, _TPW)], i1_v)
        pltpu.sync_copy(x_hbm.at[pl.ds(base, _TPW)], rows_v)
        c0 = pltpu.async_copy(rows_v, xd_hbm.at[i0_v], sem0)
        c1 = pltpu.async_copy(rows_v, xd_hbm.at[i1_v], sem1)
        c0.wait()
        c1.wait()
    return k


@functools.lru_cache(maxsize=None)
def _build_sc_combine():
    @functools.partial(
        pl.kernel,
        mesh=plsc.VectorSubcoreMesh(core_axis_name="c", subcore_axis_name="s"),
        out_type=jax.ShapeDtypeStruct((N, OUT), jnp.float32),
        scratch_types=[
            pltpu.VMEM((_TPW,), jnp.int32),
            pltpu.VMEM((_TPW,), jnp.int32),
            pltpu.VMEM((_TPW, OUT), jnp.float32),
            pltpu.VMEM((_TPW, OUT), jnp.float32),
            pltpu.SemaphoreType.DMA,
        ],
    )
    def k(outw_hbm, d0_hbm, d1_hbm, y_hbm, i0_v, i1_v, r0_v, r1_v, sem):
        wid = lax.axis_index("s") * NC + lax.axis_index("c")
        base = wid * _TPW
        pltpu.sync_copy(d0_hbm.at[pl.ds(base, _TPW)], i0_v)
        pltpu.sync_copy(d1_hbm.at[pl.ds(base, _TPW)], i1_v)
        pltpu.async_copy(outw_hbm.at[i0_v], r0_v, sem).wait()
        pltpu.async_copy(outw_hbm.at[i1_v], r1_v, sem).wait()

        def body(t, _):
            def cbody(j, _):
                cs = pl.ds(j * 16, 16)
                r0_v[t, cs] = r0_v[t, cs] + r1_v[t, cs]
                return 0
            return lax.fori_loop(0, OUT // 16, cbody, 0)

        lax.fori_loop(0, _TPW, body, 0)
        pltpu.sync_copy(r0_v, y_hbm.at[pl.ds(base, _TPW)])
    return k


def _sc_dispatch(x, dest0, dest1):
    return _build_sc_dispatch()(x, dest0, dest1)


def _sc_combine(outw, dest0, dest1):
    return _build_sc_combine()(outw, dest0, dest1)


def kernel(x, band_indices, w_gate, fc1_W, fc1_b, fc2_W, fc2_b,
           lora1_A, lora1_B, lora2_A, lora2_B):
    a1, a2, g1, g2, r0, r1, counts, loss = pl.pallas_call(
        _gating_kernel,
        out_shape=(
            jax.ShapeDtypeStruct((N, 1), jnp.int32),
            jax.ShapeDtypeStruct((N, 1), jnp.int32),
            jax.ShapeDtypeStruct((N, 1), jnp.float32),
            jax.ShapeDtypeStruct((N, 1), jnp.float32),
            jax.ShapeDtypeStruct((N, 1), jnp.int32),
            jax.ShapeDtypeStruct((N, 1), jnp.int32),
            jax.ShapeDtypeStruct((1, E), jnp.int32),
            jax.ShapeDtypeStruct((1, 1), jnp.float32),
        ),
        in_specs=[
            pl.BlockSpec((N, IN), lambda: (0, 0)),
            pl.BlockSpec((IN, E), lambda: (0, 0)),
        ],
        out_specs=(
            pl.BlockSpec((N, 1), lambda: (0, 0)),
            pl.BlockSpec((N, 1), lambda: (0, 0)),
            pl.BlockSpec((N, 1), lambda: (0, 0)),
            pl.BlockSpec((N, 1), lambda: (0, 0)),
            pl.BlockSpec((N, 1), lambda: (0, 0)),
            pl.BlockSpec((N, 1), lambda: (0, 0)),
            pl.BlockSpec((1, E), lambda: (0, 0)),
            pl.BlockSpec(memory_space=pltpu.SMEM),
        ),
    )(x, w_gate)

    counts = counts.reshape(E)
    nb = (counts + (BT - 1)) // BT
    ends = jnp.cumsum(nb)
    block_off = ends - nb
    pad_off = block_off * BT
    total_blocks = ends[E - 1]

    a1f_ = a1.reshape(N)
    a2f_ = a2.reshape(N)
    dest0 = pad_off[a1f_] + r0.reshape(N)
    dest1 = pad_off[a2f_] + r1.reshape(N)

    bj = jnp.arange(MAXB, dtype=jnp.int32)
    be_raw = jnp.sum((ends[None, :] <= bj[:, None]).astype(jnp.int32), axis=1)
    be_last = be_raw[jnp.maximum(total_blocks - 1, 0)]
    block_expert = jnp.where(bj < total_blocks, be_raw, be_last)

    bands = band_indices.astype(jnp.int32)
    gv = jnp.zeros((P,), jnp.float32).at[dest0].set(g1.reshape(N)).at[dest1].set(g2.reshape(N))
    bv = jnp.zeros((P,), jnp.int32).at[dest0].set(bands).at[dest1].set(bands)

    x_bf = x.astype(jnp.bfloat16)
    x_pack = lax.bitcast_convert_type(
        jnp.stack([x_bf[:, :_PW], x_bf[:, _PW:]], axis=-1), jnp.float32)
    xd = _sc_dispatch(x_pack, dest0, dest1)

    a1f = lora1_A.transpose(0, 2, 1, 3).reshape(E, IN, NB * R)
    bb1f = lora1_B.reshape(E, NB * R, HID)
    a2f = lora2_A.transpose(0, 2, 1, 3).reshape(E, HID, NB * R)
    bb2f = lora2_B.reshape(E, NB * R, OUT)
    b1_3d = fc1_b.reshape(E, 1, HID)
    b2_3d = fc2_b.reshape(E, 1, OUT)
    bv3 = bv.reshape(MAXB, BT, 1)
    gv3 = gv.reshape(MAXB, BT, 1)

    grid_spec = pltpu.PrefetchScalarGridSpec(
        num_scalar_prefetch=2,
        grid=(MAXB,),
        in_specs=[
            pl.BlockSpec((BT, _PW), lambda i, be, nb: (i, 0)),
            pl.BlockSpec((1, BT, 1), lambda i, be, nb: (i, 0, 0)),
            pl.BlockSpec((1, BT, 1), lambda i, be, nb: (i, 0, 0)),
            pl.BlockSpec((1, IN, HID), lambda i, be, nb: (be[i], 0, 0)),
            pl.BlockSpec((1, 1, HID), lambda i, be, nb: (be[i], 0, 0)),
            pl.BlockSpec((1, HID, OUT), lambda i, be, nb: (be[i], 0, 0)),
            pl.BlockSpec((1, 1, OUT), lambda i, be, nb: (be[i], 0, 0)),
            pl.BlockSpec((1, IN, NB * R), lambda i, be, nb: (be[i], 0, 0)),
            pl.BlockSpec((1, NB * R, HID), lambda i, be, nb: (be[i], 0, 0)),
            pl.BlockSpec((1, HID, NB * R), lambda i, be, nb: (be[i], 0, 0)),
            pl.BlockSpec((1, NB * R, OUT), lambda i, be, nb: (be[i], 0, 0)),
        ],
        out_specs=pl.BlockSpec((BT, OUT), lambda i, be, nb: (i, 0)),
    )
    outw = pl.pallas_call(
        _gmm_kernel,
        grid_spec=grid_spec,
        out_shape=jax.ShapeDtypeStruct((P, OUT), jnp.float32),
    )(block_expert, total_blocks.reshape(1), xd, bv3, gv3, fc1_W, b1_3d,
      fc2_W, b2_3d, a1f, bb1f, a2f, bb2f)

    y = _sc_combine(outw, dest0, dest1)

    return y, loss[0, 0]


# speedup vs baseline: 2.6908x; 1.0628x over previous
import functools

import jax
import jax.numpy as jnp
from jax import lax
from jax.experimental import pallas as pl
from jax.experimental.pallas import tpu as pltpu
from jax.experimental.pallas import tpu_sc as plsc

E = 64
IN = 768
HID = 1536
OUT = 768
NB = 8
R = 8
ALPHA = 16.0
K = 2
N = 2048
SCALING = ALPHA / R

BT = 128
MAXB = N * K // BT + E
P = MAXB * BT

NEG = -3.0e38

NC = 2
NS = 16
NW = NC * NS


def _gating_kernel(x_ref, wg_ref, a1_ref, a2_ref, g1_ref, g2_ref,
                   r0_ref, r1_ref, counts_ref, loss_ref):
    x = x_ref[...]
    logits = jnp.dot(x, wg_ref[...], preferred_element_type=jnp.float32)
    iota = lax.broadcasted_iota(jnp.int32, (N, E), 1)
    m1 = jnp.max(logits, axis=1, keepdims=True)
    idx1 = jnp.min(jnp.where(logits == m1, iota, E), axis=1, keepdims=True)
    sel1 = iota == idx1
    l2 = jnp.where(sel1, NEG, logits)
    m2 = jnp.max(l2, axis=1, keepdims=True)
    idx2 = jnp.min(jnp.where(l2 == m2, iota, E), axis=1, keepdims=True)
    sel2 = iota == idx2
    ed = jnp.exp(m2 - m1)
    g1 = 1.0 / (1.0 + ed)
    g2 = ed / (1.0 + ed)

    a1_ref[...] = idx1
    a2_ref[...] = idx2
    g1_ref[...] = g1
    g2_ref[...] = g2

    oh1 = sel1.astype(jnp.float32)
    oh2 = sel2.astype(jnp.float32)

    ri = lax.broadcasted_iota(jnp.int32, (N, N), 0)
    ci = lax.broadcasted_iota(jnp.int32, (N, N), 1)
    lt = (ci < ri).astype(jnp.float32)
    oh = jnp.concatenate([oh1, oh2], axis=1)
    prefix = jnp.dot(lt, oh, preferred_element_type=jnp.float32)
    p1 = prefix[:, :E]
    p2 = prefix[:, E:]
    c1 = jnp.sum(oh1, axis=0, keepdims=True)
    rank0 = jnp.sum(jnp.where(sel1, p1, 0.0), axis=1, keepdims=True)
    rank1 = jnp.sum(jnp.where(sel2, c1 + p2, 0.0), axis=1, keepdims=True)
    r0_ref[...] = rank0.astype(jnp.int32)
    r1_ref[...] = rank1.astype(jnp.int32)
    counts_ref[...] = (c1 + jnp.sum(oh2, axis=0, keepdims=True)).astype(jnp.int32)

    gates = jnp.where(sel1, g1, 0.0) + jnp.where(sel2, g2, 0.0)
    importance = jnp.sum(gates, axis=0)
    load = jnp.sum((gates > 0).astype(jnp.float32), axis=0)

    def cv_sq(v):
        mean = jnp.mean(v)
        var = jnp.sum((v - mean) ** 2) / (E - 1)
        return var / (mean * mean + 1e-10)

    loss_ref[0, 0] = (cv_sq(importance) + cv_sq(load)) * 0.01


def _gmm_kernel(be_ref, nb_ref, xd_ref, bv_ref, gv_ref,
                w1_ref, b1_ref, w2_ref, b2_ref,
                a1_ref, bb1_ref, a2_ref, bb2_ref, out_ref):
    @pl.when(pl.program_id(0) < nb_ref[0])
    def _():
        u = lax.bitcast_convert_type(xd_ref[...], jnp.uint32)
        xa = lax.bitcast_convert_type(u << 16, jnp.float32)
        xb = lax.bitcast_convert_type((u >> 16) << 16, jnp.float32)
        x = jnp.concatenate([xa, xb], axis=1)
        bands = bv_ref[0]
        iota_nbr = lax.broadcasted_iota(jnp.int32, (BT, NB * R), 1)
        mask = (lax.div(iota_nbr, R) == bands).astype(jnp.float32)

        lh = jnp.dot(x, a1_ref[0], preferred_element_type=jnp.float32) * mask
        lh = jnp.dot(lh, bb1_ref[0], preferred_element_type=jnp.float32)
        h = jnp.dot(x, w1_ref[0], preferred_element_type=jnp.float32)
        h = h + b1_ref[0] + lh * SCALING
        h = h * 0.5 * (1.0 + lax.erf(h * 0.7071067811865476))

        lo = jnp.dot(h, a2_ref[0], preferred_element_type=jnp.float32) * mask
        lo = jnp.dot(lo, bb2_ref[0], preferred_element_type=jnp.float32)
        out = jnp.dot(h, w2_ref[0], preferred_element_type=jnp.float32)
        out = out + b2_ref[0] + lo * SCALING
        out_ref[...] = out * gv_ref[0]


_CH = 64
_PW = IN // 2
_RPW = P // NW
_TPW = N // NW


@functools.lru_cache(maxsize=None)
def _build_sc_dispatch():
    @functools.partial(
        pl.kernel,
        mesh=plsc.VectorSubcoreMesh(core_axis_name="c", subcore_axis_name="s"),
        out_type=(
            jax.ShapeDtypeStruct((P, _PW), jnp.float32),
            jax.ShapeDtypeStruct((P,), jnp.float32),
            jax.ShapeDtypeStruct((P,), jnp.int32),
        ),
        scratch_types=[
            pltpu.VMEM((_TPW,), jnp.int32),
            pltpu.VMEM((_TPW,), jnp.int32),
            pltpu.VMEM((_TPW, _PW), jnp.float32),
            pltpu.VMEM((_TPW,), jnp.float32),
            pltpu.VMEM((_TPW,), jnp.float32),
            pltpu.VMEM((_TPW,), jnp.int32),
        ] + [pltpu.SemaphoreType.DMA] * 6,
    )
    def k(x_hbm, d0_hbm, d1_hbm, g1_hbm, g2_hbm, bd_hbm,
          xd_hbm, gv_hbm, bv_hbm,
          i0_v, i1_v, rows_v, g1_v, g2_v, bd_v, *sems):
        wid = lax.axis_index("s") * NC + lax.axis_index("c")
        base = wid * _TPW
        pltpu.sync_copy(d0_hbm.at[pl.ds(base, _TPW)], i0_v)
        pltpu.sync_copy(d1_hbm.at[pl.ds(base, _TPW)], i1_v)
        pltpu.sync_copy(x_hbm.at[pl.ds(base, _TPW)], rows_v)
        pltpu.sync_copy(g1_hbm.at[pl.ds(base, _TPW)], g1_v)
        pltpu.sync_copy(g2_hbm.at[pl.ds(base, _TPW)], g2_v)
        pltpu.sync_copy(bd_hbm.at[pl.ds(base, _TPW)], bd_v)
        copies = [
            pltpu.async_copy(rows_v, xd_hbm.at[i0_v], sems[0]),
            pltpu.async_copy(rows_v, xd_hbm.at[i1_v], sems[1]),
            pltpu.async_copy(g1_v, gv_hbm.at[i0_v], sems[2]),
            pltpu.async_copy(g2_v, gv_hbm.at[i1_v], sems[3]),
            pltpu.async_copy(bd_v, bv_hbm.at[i0_v], sems[4]),
            pltpu.async_copy(bd_v, bv_hbm.at[i1_v], sems[5]),
        ]
        for c in copies:
            c.wait()
    return k


@functools.lru_cache(maxsize=None)
def _build_sc_combine():
    @functools.partial(
        pl.kernel,
        mesh=plsc.VectorSubcoreMesh(core_axis_name="c", subcore_axis_name="s"),
        out_type=jax.ShapeDtypeStruct((N, OUT), jnp.float32),
        scratch_types=[
            pltpu.VMEM((_TPW,), jnp.int32),
            pltpu.VMEM((_TPW,), jnp.int32),
            pltpu.VMEM((_TPW, OUT), jnp.float32),
            pltpu.VMEM((_TPW, OUT), jnp.float32),
            pltpu.SemaphoreType.DMA,
        ],
    )
    def k(outw_hbm, d0_hbm, d1_hbm, y_hbm, i0_v, i1_v, r0_v, r1_v, sem):
        wid = lax.axis_index("s") * NC + lax.axis_index("c")
        base = wid * _TPW
        pltpu.sync_copy(d0_hbm.at[pl.ds(base, _TPW)], i0_v)
        pltpu.sync_copy(d1_hbm.at[pl.ds(base, _TPW)], i1_v)
        pltpu.async_copy(outw_hbm.at[i0_v], r0_v, sem).wait()
        pltpu.async_copy(outw_hbm.at[i1_v], r1_v, sem).wait()

        def body(t, _):
            def cbody(j, _):
                cs = pl.ds(j * 16, 16)
                r0_v[t, cs] = r0_v[t, cs] + r1_v[t, cs]
                return 0
            return lax.fori_loop(0, OUT // 16, cbody, 0)

        lax.fori_loop(0, _TPW, body, 0)
        pltpu.sync_copy(r0_v, y_hbm.at[pl.ds(base, _TPW)])
    return k


def _sc_dispatch(x, dest0, dest1, g1, g2, bands):
    return _build_sc_dispatch()(x, dest0, dest1, g1, g2, bands)


def _sc_combine(outw, dest0, dest1):
    return _build_sc_combine()(outw, dest0, dest1)


def kernel(x, band_indices, w_gate, fc1_W, fc1_b, fc2_W, fc2_b,
           lora1_A, lora1_B, lora2_A, lora2_B):
    a1, a2, g1, g2, r0, r1, counts, loss = pl.pallas_call(
        _gating_kernel,
        out_shape=(
            jax.ShapeDtypeStruct((N, 1), jnp.int32),
            jax.ShapeDtypeStruct((N, 1), jnp.int32),
            jax.ShapeDtypeStruct((N, 1), jnp.float32),
            jax.ShapeDtypeStruct((N, 1), jnp.float32),
            jax.ShapeDtypeStruct((N, 1), jnp.int32),
            jax.ShapeDtypeStruct((N, 1), jnp.int32),
            jax.ShapeDtypeStruct((1, E), jnp.int32),
            jax.ShapeDtypeStruct((1, 1), jnp.float32),
        ),
        in_specs=[
            pl.BlockSpec((N, IN), lambda: (0, 0)),
            pl.BlockSpec((IN, E), lambda: (0, 0)),
        ],
        out_specs=(
            pl.BlockSpec((N, 1), lambda: (0, 0)),
            pl.BlockSpec((N, 1), lambda: (0, 0)),
            pl.BlockSpec((N, 1), lambda: (0, 0)),
            pl.BlockSpec((N, 1), lambda: (0, 0)),
            pl.BlockSpec((N, 1), lambda: (0, 0)),
            pl.BlockSpec((N, 1), lambda: (0, 0)),
            pl.BlockSpec((1, E), lambda: (0, 0)),
            pl.BlockSpec(memory_space=pltpu.SMEM),
        ),
    )(x, w_gate)

    counts = counts.reshape(E)
    nb = (counts + (BT - 1)) // BT
    ends = jnp.cumsum(nb)
    block_off = ends - nb
    pad_off = block_off * BT
    total_blocks = ends[E - 1]

    a1f_ = a1.reshape(N)
    a2f_ = a2.reshape(N)
    dest0 = pad_off[a1f_] + r0.reshape(N)
    dest1 = pad_off[a2f_] + r1.reshape(N)

    bj = jnp.arange(MAXB, dtype=jnp.int32)
    be_raw = jnp.sum((ends[None, :] <= bj[:, None]).astype(jnp.int32), axis=1)
    be_last = be_raw[jnp.maximum(total_blocks - 1, 0)]
    block_expert = jnp.where(bj < total_blocks, be_raw, be_last)

    bands = band_indices.astype(jnp.int32)

    x_bf = x.astype(jnp.bfloat16)
    x_pack = lax.bitcast_convert_type(
        jnp.stack([x_bf[:, :_PW], x_bf[:, _PW:]], axis=-1), jnp.float32)
    xd, gv, bv = _sc_dispatch(x_pack, dest0, dest1,
                              g1.reshape(N), g2.reshape(N), bands)

    a1f = lora1_A.transpose(0, 2, 1, 3).reshape(E, IN, NB * R)
    bb1f = lora1_B.reshape(E, NB * R, HID)
    a2f = lora2_A.transpose(0, 2, 1, 3).reshape(E, HID, NB * R)
    bb2f = lora2_B.reshape(E, NB * R, OUT)
    b1_3d = fc1_b.reshape(E, 1, HID)
    b2_3d = fc2_b.reshape(E, 1, OUT)
    bv3 = bv.reshape(MAXB, BT, 1)
    gv3 = gv.reshape(MAXB, BT, 1)

    grid_spec = pltpu.PrefetchScalarGridSpec(
        num_scalar_prefetch=2,
        grid=(MAXB,),
        in_specs=[
            pl.BlockSpec((BT, _PW), lambda i, be, nb: (i, 0)),
            pl.BlockSpec((1, BT, 1), lambda i, be, nb: (i, 0, 0)),
            pl.BlockSpec((1, BT, 1), lambda i, be, nb: (i, 0, 0)),
            pl.BlockSpec((1, IN, HID), lambda i, be, nb: (be[i], 0, 0)),
            pl.BlockSpec((1, 1, HID), lambda i, be, nb: (be[i], 0, 0)),
            pl.BlockSpec((1, HID, OUT), lambda i, be, nb: (be[i], 0, 0)),
            pl.BlockSpec((1, 1, OUT), lambda i, be, nb: (be[i], 0, 0)),
            pl.BlockSpec((1, IN, NB * R), lambda i, be, nb: (be[i], 0, 0)),
            pl.BlockSpec((1, NB * R, HID), lambda i, be, nb: (be[i], 0, 0)),
            pl.BlockSpec((1, HID, NB * R), lambda i, be, nb: (be[i], 0, 0)),
            pl.BlockSpec((1, NB * R, OUT), lambda i, be, nb: (be[i], 0, 0)),
        ],
        out_specs=pl.BlockSpec((BT, OUT), lambda i, be, nb: (i, 0)),
    )
    outw = pl.pallas_call(
        _gmm_kernel,
        grid_spec=grid_spec,
        out_shape=jax.ShapeDtypeStruct((P, OUT), jnp.float32),
    )(block_expert, total_blocks.reshape(1), xd, bv3, gv3, fc1_W, b1_3d,
      fc2_W, b2_3d, a1f, bb1f, a2f, bb2f)

    y = _sc_combine(outw, dest0, dest1)

    return y, loss[0, 0]


# speedup vs baseline: 3.5140x; 1.3059x over previous
import functools

import jax
import jax.numpy as jnp
from jax import lax
from jax.experimental import pallas as pl
from jax.experimental.pallas import tpu as pltpu
from jax.experimental.pallas import tpu_sc as plsc

E = 64
IN = 768
HID = 1536
OUT = 768
NB = 8
R = 8
ALPHA = 16.0
K = 2
N = 2048
SCALING = ALPHA / R

BT = 128
MAXB = N * K // BT + E
P = MAXB * BT

NEG = -3.0e38

NC = 2
NS = 16
NW = NC * NS


def _gating_kernel(x_ref, wg_ref, xp_ref, d0_ref, d1_ref, g1_ref, g2_ref,
                   be_ref, nb_ref, loss_ref):
    x = x_ref[...]

    au = lax.bitcast_convert_type(
        x[:, :_PW].astype(jnp.bfloat16), jnp.uint16).astype(jnp.uint32)
    bu = lax.bitcast_convert_type(
        x[:, _PW:].astype(jnp.bfloat16), jnp.uint16).astype(jnp.uint32)
    xp_ref[...] = lax.bitcast_convert_type(au | (bu << 16), jnp.float32)
    logits = jnp.dot(x, wg_ref[...], preferred_element_type=jnp.float32)
    iota = lax.broadcasted_iota(jnp.int32, (N, E), 1)
    m1 = jnp.max(logits, axis=1, keepdims=True)
    idx1 = jnp.min(jnp.where(logits == m1, iota, E), axis=1, keepdims=True)
    sel1 = iota == idx1
    l2 = jnp.where(sel1, NEG, logits)
    m2 = jnp.max(l2, axis=1, keepdims=True)
    idx2 = jnp.min(jnp.where(l2 == m2, iota, E), axis=1, keepdims=True)
    sel2 = iota == idx2
    ed = jnp.exp(m2 - m1)
    g1 = 1.0 / (1.0 + ed)
    g2 = ed / (1.0 + ed)

    g1_ref[...] = g1
    g2_ref[...] = g2

    oh1 = sel1.astype(jnp.float32)
    oh2 = sel2.astype(jnp.float32)

    ri = lax.broadcasted_iota(jnp.int32, (N, N), 0)
    ci = lax.broadcasted_iota(jnp.int32, (N, N), 1)
    lt = (ci < ri).astype(jnp.float32)
    oh = jnp.concatenate([oh1, oh2], axis=1)
    prefix = jnp.dot(lt, oh, preferred_element_type=jnp.float32)
    p1 = prefix[:, :E]
    p2 = prefix[:, E:]
    c1 = jnp.sum(oh1, axis=0, keepdims=True)
    rank0 = jnp.sum(jnp.where(sel1, p1, 0.0), axis=1, keepdims=True)
    rank1 = jnp.sum(jnp.where(sel2, c1 + p2, 0.0), axis=1, keepdims=True)
    counts = c1 + jnp.sum(oh2, axis=0, keepdims=True)

    nb = jnp.floor((counts + (BT - 1)) * (1.0 / BT))
    ei = lax.broadcasted_iota(jnp.int32, (E, E), 0)
    ej = lax.broadcasted_iota(jnp.int32, (E, E), 1)
    ends = jnp.dot(nb, (ei <= ej).astype(jnp.float32),
                   preferred_element_type=jnp.float32)
    pad_off = (ends - nb) * BT

    d0 = jnp.sum(jnp.where(sel1, pad_off, 0.0), axis=1, keepdims=True) + rank0
    d1 = jnp.sum(jnp.where(sel2, pad_off, 0.0), axis=1, keepdims=True) + rank1
    d0_ref[...] = d0.astype(jnp.int32)
    d1_ref[...] = d1.astype(jnp.int32)

    bj = lax.broadcasted_iota(jnp.int32, (MAXB, 1), 0).astype(jnp.float32)
    total = jnp.sum(nb)
    be_raw = jnp.sum((jnp.broadcast_to(ends, (MAXB, E)) <= bj)
                     .astype(jnp.float32), axis=1, keepdims=True)
    be_last = jnp.sum(jnp.where(bj == total - 1.0, be_raw, 0.0))
    be = jnp.where(bj < total, be_raw, be_last)
    be_ref[...] = be.astype(jnp.int32)
    nb_ref[0, 0] = total.astype(jnp.int32)

    gates = jnp.where(sel1, g1, 0.0) + jnp.where(sel2, g2, 0.0)
    importance = jnp.sum(gates, axis=0)
    load = jnp.sum((gates > 0).astype(jnp.float32), axis=0)

    def cv_sq(v):
        mean = jnp.mean(v)
        var = jnp.sum((v - mean) ** 2) / (E - 1)
        return var / (mean * mean + 1e-10)

    loss_ref[0, 0] = (cv_sq(importance) + cv_sq(load)) * 0.01


def _gmm_kernel(be_ref, nb_ref, xd_ref, bv_ref, gv_ref,
                w1_ref, b1_ref, w2_ref, b2_ref,
                a1_ref, bb1_ref, a2_ref, bb2_ref, out_ref):
    @pl.when(pl.program_id(0) < nb_ref[0])
    def _():
        u = lax.bitcast_convert_type(xd_ref[...], jnp.uint32)
        xa = lax.bitcast_convert_type(u << 16, jnp.float32)
        xb = lax.bitcast_convert_type((u >> 16) << 16, jnp.float32)
        x = jnp.concatenate([xa, xb], axis=1)
        bands = bv_ref[0]
        iota_nbr = lax.broadcasted_iota(jnp.int32, (BT, NB * R), 1)
        mask = (lax.div(iota_nbr, R) == bands).astype(jnp.float32)

        lh = jnp.dot(x, a1_ref[0], preferred_element_type=jnp.float32) * mask
        lh = jnp.dot(lh, bb1_ref[0], preferred_element_type=jnp.float32)
        h = jnp.dot(x, w1_ref[0], preferred_element_type=jnp.float32)
        h = h + b1_ref[0] + lh * SCALING
        h = h * 0.5 * (1.0 + lax.erf(h * 0.7071067811865476))

        lo = jnp.dot(h, a2_ref[0], preferred_element_type=jnp.float32) * mask
        lo = jnp.dot(lo, bb2_ref[0], preferred_element_type=jnp.float32)
        out = jnp.dot(h, w2_ref[0], preferred_element_type=jnp.float32)
        out = out + b2_ref[0] + lo * SCALING
        out_ref[...] = out * gv_ref[0]


_CH = 64
_PW = IN // 2
_RPW = P // NW
_TPW = N // NW


@functools.lru_cache(maxsize=None)
def _build_sc_dispatch():
    @functools.partial(
        pl.kernel,
        mesh=plsc.VectorSubcoreMesh(core_axis_name="c", subcore_axis_name="s"),
        out_type=(
            jax.ShapeDtypeStruct((P, _PW), jnp.float32),
            jax.ShapeDtypeStruct((P,), jnp.float32),
            jax.ShapeDtypeStruct((P,), jnp.int32),
        ),
        scratch_types=[
            pltpu.VMEM((_TPW,), jnp.int32),
            pltpu.VMEM((_TPW,), jnp.int32),
            pltpu.VMEM((_TPW, _PW), jnp.float32),
            pltpu.VMEM((_TPW,), jnp.float32),
            pltpu.VMEM((_TPW,), jnp.float32),
            pltpu.VMEM((_TPW,), jnp.int32),
        ] + [pltpu.SemaphoreType.DMA] * 6,
    )
    def k(x_hbm, d0_hbm, d1_hbm, g1_hbm, g2_hbm, bd_hbm,
          xd_hbm, gv_hbm, bv_hbm,
          i0_v, i1_v, rows_v, g1_v, g2_v, bd_v, *sems):
        wid = lax.axis_index("s") * NC + lax.axis_index("c")
        base = wid * _TPW
        pltpu.sync_copy(d0_hbm.at[pl.ds(base, _TPW)], i0_v)
        pltpu.sync_copy(d1_hbm.at[pl.ds(base, _TPW)], i1_v)
        pltpu.sync_copy(x_hbm.at[pl.ds(base, _TPW)], rows_v)
        pltpu.sync_copy(g1_hbm.at[pl.ds(base, _TPW)], g1_v)
        pltpu.sync_copy(g2_hbm.at[pl.ds(base, _TPW)], g2_v)
        pltpu.sync_copy(bd_hbm.at[pl.ds(base, _TPW)], bd_v)
        copies = [
            pltpu.async_copy(rows_v, xd_hbm.at[i0_v], sems[0]),
            pltpu.async_copy(rows_v, xd_hbm.at[i1_v], sems[1]),
            pltpu.async_copy(g1_v, gv_hbm.at[i0_v], sems[2]),
            pltpu.async_copy(g2_v, gv_hbm.at[i1_v], sems[3]),
            pltpu.async_copy(bd_v, bv_hbm.at[i0_v], sems[4]),
            pltpu.async_copy(bd_v, bv_hbm.at[i1_v], sems[5]),
        ]
        for c in copies:
            c.wait()
    return k


@functools.lru_cache(maxsize=None)
def _build_sc_combine():
    @functools.partial(
        pl.kernel,
        mesh=plsc.VectorSubcoreMesh(core_axis_name="c", subcore_axis_name="s"),
        out_type=jax.ShapeDtypeStruct((N, OUT), jnp.float32),
        scratch_types=[
            pltpu.VMEM((_TPW,), jnp.int32),
            pltpu.VMEM((_TPW,), jnp.int32),
            pltpu.VMEM((_TPW, OUT), jnp.float32),
            pltpu.VMEM((_TPW, OUT), jnp.float32),
            pltpu.SemaphoreType.DMA,
        ],
    )
    def k(outw_hbm, d0_hbm, d1_hbm, y_hbm, i0_v, i1_v, r0_v, r1_v, sem):
        wid = lax.axis_index("s") * NC + lax.axis_index("c")
        base = wid * _TPW
        pltpu.sync_copy(d0_hbm.at[pl.ds(base, _TPW)], i0_v)
        pltpu.sync_copy(d1_hbm.at[pl.ds(base, _TPW)], i1_v)
        pltpu.async_copy(outw_hbm.at[i0_v], r0_v, sem).wait()
        pltpu.async_copy(outw_hbm.at[i1_v], r1_v, sem).wait()

        def body(t, _):
            def cbody(j, _):
                cs = pl.ds(j * 16, 16)
                r0_v[t, cs] = r0_v[t, cs] + r1_v[t, cs]
                return 0
            return lax.fori_loop(0, OUT // 16, cbody, 0)

        lax.fori_loop(0, _TPW, body, 0)
        pltpu.sync_copy(r0_v, y_hbm.at[pl.ds(base, _TPW)])
    return k


def _sc_dispatch(x, dest0, dest1, g1, g2, bands):
    return _build_sc_dispatch()(x, dest0, dest1, g1, g2, bands)


def _sc_combine(outw, dest0, dest1):
    return _build_sc_combine()(outw, dest0, dest1)


def kernel(x, band_indices, w_gate, fc1_W, fc1_b, fc2_W, fc2_b,
           lora1_A, lora1_B, lora2_A, lora2_B):
    x_pack, d0, d1, g1, g2, be, nb_tot, loss = pl.pallas_call(
        _gating_kernel,
        out_shape=(
            jax.ShapeDtypeStruct((N, _PW), jnp.float32),
            jax.ShapeDtypeStruct((N, 1), jnp.int32),
            jax.ShapeDtypeStruct((N, 1), jnp.int32),
            jax.ShapeDtypeStruct((N, 1), jnp.float32),
            jax.ShapeDtypeStruct((N, 1), jnp.float32),
            jax.ShapeDtypeStruct((MAXB, 1), jnp.int32),
            jax.ShapeDtypeStruct((1, 1), jnp.int32),
            jax.ShapeDtypeStruct((1, 1), jnp.float32),
        ),
        in_specs=[
            pl.BlockSpec((N, IN), lambda: (0, 0)),
            pl.BlockSpec((IN, E), lambda: (0, 0)),
        ],
        out_specs=(
            pl.BlockSpec((N, _PW), lambda: (0, 0)),
            pl.BlockSpec((N, 1), lambda: (0, 0)),
            pl.BlockSpec((N, 1), lambda: (0, 0)),
            pl.BlockSpec((N, 1), lambda: (0, 0)),
            pl.BlockSpec((N, 1), lambda: (0, 0)),
            pl.BlockSpec((MAXB, 1), lambda: (0, 0)),
            pl.BlockSpec(memory_space=pltpu.SMEM),
            pl.BlockSpec(memory_space=pltpu.SMEM),
        ),
    )(x, w_gate)

    dest0 = d0.reshape(N)
    dest1 = d1.reshape(N)
    block_expert = be.reshape(MAXB)
    total_blocks = nb_tot.reshape(1)
    bands = band_indices.astype(jnp.int32)

    xd, gv, bv = _sc_dispatch(x_pack, dest0, dest1,
                              g1.reshape(N), g2.reshape(N), bands)

    a1f = lora1_A.transpose(0, 2, 1, 3).reshape(E, IN, NB * R)
    bb1f = lora1_B.reshape(E, NB * R, HID)
    a2f = lora2_A.transpose(0, 2, 1, 3).reshape(E, HID, NB * R)
    bb2f = lora2_B.reshape(E, NB * R, OUT)
    b1_3d = fc1_b.reshape(E, 1, HID)
    b2_3d = fc2_b.reshape(E, 1, OUT)
    bv3 = bv.reshape(MAXB, BT, 1)
    gv3 = gv.reshape(MAXB, BT, 1)

    grid_spec = pltpu.PrefetchScalarGridSpec(
        num_scalar_prefetch=2,
        grid=(MAXB,),
        in_specs=[
            pl.BlockSpec((BT, _PW), lambda i, be, nb: (i, 0)),
            pl.BlockSpec((1, BT, 1), lambda i, be, nb: (i, 0, 0)),
            pl.BlockSpec((1, BT, 1), lambda i, be, nb: (i, 0, 0)),
            pl.BlockSpec((1, IN, HID), lambda i, be, nb: (be[i], 0, 0)),
            pl.BlockSpec((1, 1, HID), lambda i, be, nb: (be[i], 0, 0)),
            pl.BlockSpec((1, HID, OUT), lambda i, be, nb: (be[i], 0, 0)),
            pl.BlockSpec((1, 1, OUT), lambda i, be, nb: (be[i], 0, 0)),
            pl.BlockSpec((1, IN, NB * R), lambda i, be, nb: (be[i], 0, 0)),
            pl.BlockSpec((1, NB * R, HID), lambda i, be, nb: (be[i], 0, 0)),
            pl.BlockSpec((1, HID, NB * R), lambda i, be, nb: (be[i], 0, 0)),
            pl.BlockSpec((1, NB * R, OUT), lambda i, be, nb: (be[i], 0, 0)),
        ],
        out_specs=pl.BlockSpec((BT, OUT), lambda i, be, nb: (i, 0)),
    )
    outw = pl.pallas_call(
        _gmm_kernel,
        grid_spec=grid_spec,
        out_shape=jax.ShapeDtypeStruct((P, OUT), jnp.float32),
    )(block_expert, total_blocks, xd, bv3, gv3, fc1_W, b1_3d,
      fc2_W, b2_3d, a1f, bb1f, a2f, bb2f)

    y = _sc_combine(outw, dest0, dest1)

    return y, loss[0, 0]
